# Initial kernel scaffold; baseline (speedup 1.0000x reference)
import jax
import jax.numpy as jnp
from jax.experimental import pallas as pl


def kernel(nodes_real, nodes, samp_neighs, feature_table):
    raise NotImplementedError("write your pallas kernel here")


# speedup vs baseline: 1.7080x; 1.7080x over previous
import functools

import jax
import jax.numpy as jnp
from jax import lax
from jax.experimental import pallas as pl
from jax.experimental.pallas import tpu as pltpu
from jax.experimental.pallas import tpu_sc as plsc

L = 16
NW = 32
S = 32
D = 128
ROWS_PER_W = 320
PAD_B = NW * ROWS_PER_W


_DG_DNUMS = lax.GatherDimensionNumbers(
    offset_dims=(), collapsed_slice_dims=(0,), start_index_map=(0,))


def _dg(x, idx):
    return lax.gather(x, idx[:, None], _DG_DNUMS, slice_sizes=(1,),
                      mode=lax.GatherScatterMode.PROMISE_IN_BOUNDS)


def _row_weights(idx_ref, base):
    a = idx_ref[pl.ds(base, L)]
    b = idx_ref[pl.ds(base + L, L)]
    iota = lax.iota(jnp.int32, L)
    ca = jnp.ones((L,), jnp.float32)
    cb = jnp.ones((L,), jnp.float32)
    for r in range(L):
        ir = (iota + r) & (L - 1)
        ra = _dg(a, ir) if r else a
        rb = _dg(b, ir) if r else b
        if r:
            ca = jnp.where(a == ra, ca + 1.0, ca)
            cb = jnp.where(b == rb, cb + 1.0, cb)
        ca = jnp.where(a == rb, ca + 1.0, ca)
        cb = jnp.where(b == ra, cb + 1.0, cb)
    wa = 1.0 / ca
    wb = 1.0 / cb
    s = wa + wb
    for sh in (1, 2, 4, 8):
        s = s + _dg(s, (iota + sh) & (L - 1))
    inv_n = 1.0 / s
    return wa * inv_n, wb * inv_n


def _row_reduce(buf, wa, wb):
    accs = [jnp.zeros((L,), jnp.float32) for _ in range(D // L)]
    for s in range(S):
        wv = wa if s < L else wb
        ws = _dg(wv, jnp.full((L,), s % L, jnp.int32))
        for c in range(D // L):
            accs[c] = accs[c] + buf[s, pl.ds(c * L, L)] * ws
    return accs


_mesh = plsc.VectorSubcoreMesh(core_axis_name="c", subcore_axis_name="s")


@functools.partial(
    pl.kernel,
    out_type=jax.ShapeDtypeStruct((PAD_B, D), jnp.float32),
    mesh=_mesh,
    scratch_types=[
        pltpu.VMEM((ROWS_PER_W * S,), jnp.int32),
        pltpu.VMEM((S, D), jnp.float32),
        pltpu.VMEM((S, D), jnp.float32),
        pltpu.VMEM((ROWS_PER_W, D), jnp.float32),
        pltpu.SemaphoreType.DMA,
        pltpu.SemaphoreType.DMA,
    ],
)
def _sc_agg(table_hbm, idx_hbm, out_hbm, idx_v, buf0, buf1, out_v, sem0, sem1):
    wid = lax.axis_index("s") * 2 + lax.axis_index("c")
    base = pl.multiple_of(wid * ROWS_PER_W, 8)
    pltpu.sync_copy(idx_hbm.at[pl.ds(base * S, ROWS_PER_W * S)], idx_v)

    def gather_row(i, buf, sem):
        off = pl.multiple_of(i * S, S)
        pltpu.async_copy(table_hbm.at[idx_v.at[pl.ds(off, S)]], buf, sem)

    def wait_row(i, buf, sem):
        off = pl.multiple_of(i * S, S)
        pltpu.make_async_copy(
            table_hbm.at[idx_v.at[pl.ds(off, S)]], buf, sem).wait()

    def step(g, carry):
        for b, (buf, sem) in enumerate(((buf0, sem0), (buf1, sem1))):
            i = 2 * g + b
            wait_row(i, buf, sem)
            wa, wb = _row_weights(idx_v, pl.multiple_of(i * S, S))
            accs = _row_reduce(buf, wa, wb)

            @pl.when(i + 2 < ROWS_PER_W)
            def _():
                gather_row(i + 2, buf, sem)

            for c in range(D // L):
                out_v[i, pl.ds(c * L, L)] = accs[c]
        return carry

    gather_row(0, buf0, sem0)
    gather_row(1, buf1, sem1)
    lax.fori_loop(0, ROWS_PER_W // 2, step, 0)
    pltpu.sync_copy(out_v, out_hbm.at[pl.ds(base, ROWS_PER_W)])


def kernel(nodes_real, nodes, samp_neighs, feature_table):
    idx = samp_neighs.astype(jnp.int32)
    b = idx.shape[0]
    idx_flat = jnp.pad(idx, ((0, PAD_B - b), (0, 0))).reshape(-1)
    out = _sc_agg(feature_table, idx_flat)
    return out[:b]


# speedup vs baseline: 1.7186x; 1.0062x over previous
import functools

import jax
import jax.numpy as jnp
from jax import lax
from jax.experimental import pallas as pl
from jax.experimental.pallas import tpu as pltpu
from jax.experimental.pallas import tpu_sc as plsc

L = 16
NW = 32
S = 32
D = 128
ROWS_PER_W = 320
PAD_B = NW * ROWS_PER_W


_DG_DNUMS = lax.GatherDimensionNumbers(
    offset_dims=(), collapsed_slice_dims=(0,), start_index_map=(0,))


def _dg(x, idx):
    return lax.gather(x, idx[:, None], _DG_DNUMS, slice_sizes=(1,),
                      mode=lax.GatherScatterMode.PROMISE_IN_BOUNDS)


def _row_weights(idx_ref, base):
    a = idx_ref[pl.ds(base, L)]
    b = idx_ref[pl.ds(base + L, L)]
    iota = lax.iota(jnp.int32, L)
    ca = jnp.ones((L,), jnp.float32)
    cb = jnp.ones((L,), jnp.float32)
    for r in range(L):
        ir = (iota + r) & (L - 1)
        ra = _dg(a, ir) if r else a
        rb = _dg(b, ir) if r else b
        if r:
            ca = jnp.where(a == ra, ca + 1.0, ca)
            cb = jnp.where(b == rb, cb + 1.0, cb)
        ca = jnp.where(a == rb, ca + 1.0, ca)
        cb = jnp.where(b == ra, cb + 1.0, cb)
    wa = 1.0 / ca
    wb = 1.0 / cb
    s = wa + wb
    for sh in (1, 2, 4, 8):
        s = s + _dg(s, (iota + sh) & (L - 1))
    inv_n = 1.0 / s
    return wa * inv_n, wb * inv_n


def _row_reduce(buf, wa, wb):
    accs = [jnp.zeros((L,), jnp.float32) for _ in range(D // L)]
    for s in range(S):
        wv = wa if s < L else wb
        ws = _dg(wv, jnp.full((L,), s % L, jnp.int32))
        for c in range(D // L):
            accs[c] = accs[c] + buf[s, pl.ds(c * L, L)] * ws
    return accs


_mesh = plsc.VectorSubcoreMesh(core_axis_name="c", subcore_axis_name="s")


@functools.partial(
    pl.kernel,
    out_type=jax.ShapeDtypeStruct((PAD_B, D), jnp.float32),
    mesh=_mesh,
    scratch_types=[
        pltpu.VMEM((ROWS_PER_W * S,), jnp.int32),
        pltpu.VMEM((S, D), jnp.float32),
        pltpu.VMEM((S, D), jnp.float32),
        pltpu.VMEM((S, D), jnp.float32),
        pltpu.VMEM((S, D), jnp.float32),
        pltpu.VMEM((ROWS_PER_W, D), jnp.float32),
        pltpu.SemaphoreType.DMA,
        pltpu.SemaphoreType.DMA,
        pltpu.SemaphoreType.DMA,
        pltpu.SemaphoreType.DMA,
    ],
)
def _sc_agg(table_hbm, idx_hbm, out_hbm, idx_v, buf0, buf1, buf2, buf3,
            out_v, sem0, sem1, sem2, sem3):
    wid = lax.axis_index("s") * 2 + lax.axis_index("c")
    base = pl.multiple_of(wid * ROWS_PER_W, 8)
    pltpu.sync_copy(idx_hbm.at[pl.ds(base * S, ROWS_PER_W * S)], idx_v)

    def gather_row(i, buf, sem):
        off = pl.multiple_of(i * S, S)
        pltpu.async_copy(table_hbm.at[idx_v.at[pl.ds(off, S)]], buf, sem)

    def wait_row(i, buf, sem):
        off = pl.multiple_of(i * S, S)
        pltpu.make_async_copy(
            table_hbm.at[idx_v.at[pl.ds(off, S)]], buf, sem).wait()

    bufs = ((buf0, sem0), (buf1, sem1), (buf2, sem2), (buf3, sem3))

    def step(g, carry):
        for b, (buf, sem) in enumerate(bufs):
            i = 4 * g + b
            wait_row(i, buf, sem)
            wa, wb = _row_weights(idx_v, pl.multiple_of(i * S, S))
            accs = _row_reduce(buf, wa, wb)

            @pl.when(i + 4 < ROWS_PER_W)
            def _():
                gather_row(i + 4, buf, sem)

            for c in range(D // L):
                out_v[i, pl.ds(c * L, L)] = accs[c]
        return carry

    for b, (buf, sem) in enumerate(bufs):
        gather_row(b, buf, sem)
    lax.fori_loop(0, ROWS_PER_W // 4, step, 0)
    pltpu.sync_copy(out_v, out_hbm.at[pl.ds(base, ROWS_PER_W)])


def kernel(nodes_real, nodes, samp_neighs, feature_table):
    idx = samp_neighs.astype(jnp.int32)
    b = idx.shape[0]
    idx_flat = jnp.pad(idx, ((0, PAD_B - b), (0, 0))).reshape(-1)
    out = _sc_agg(feature_table, idx_flat)
    return out[:b]
